# baseline (speedup 1.0000x reference)
import functools

import jax
import jax.numpy as jnp
from jax import lax
from jax.experimental import pallas as pl
from jax.experimental.pallas import tpu as pltpu
from jax.experimental.pallas import tpu_sc as plsc

N = 10000
E = 160000
HID = 128
D = 64
DW = 128

NSC = 2
NTEC = 16
LW = 128
EP = 163840
CPW = EP // (NSC * NTEC * LW)
NPAD = 10240
STRIPE = NPAD // NTEC

EB = 1024
NB = 1280
HB = 6400

_mesh = plsc.VectorSubcoreMesh(core_axis_name="c", subcore_axis_name="s")


def _node0_body(x_ref, w_ref, b_ref, o_ref):
    o_ref[...] = jax.nn.relu(
        jnp.dot(x_ref[...], w_ref[...], preferred_element_type=jnp.float32)
        + b_ref[...]
    )


def _hid_body(ea_ref, w_ref, b_ref, o_ref):
    o_ref[...] = jax.nn.relu(
        jnp.dot(ea_ref[...], w_ref[...], preferred_element_type=jnp.float32)
        + b_ref[...]
    )


def _msg_body(hid_ref, xj_ref, et_ref, bmat_ref, o_ref):
    hbT = hid_ref[...].T.astype(jnp.bfloat16)
    wft = jnp.dot(et_ref[...], hbT, preferred_element_type=jnp.float32)
    xjT = xj_ref[...].T
    acc = xjT[0:1, :] * wft[0:D, :]
    for i in range(1, D):
        acc = acc + xjT[i:i + 1, :] * wft[i * D:(i + 1) * D, :]
    msg64 = acc.T
    o_ref[...] = jnp.concatenate(
        [msg64, jnp.zeros_like(msg64)], axis=1
    ) + jnp.dot(xj_ref[...], bmat_ref[...], preferred_element_type=jnp.float32)


def _update_body(agg_ref, cnt_ref, h_ref, whh_ref, bh_ref, wtop_ref, wbot_ref,
                 bhm_ref, cb_ref, out_ref, hn_ref, outr_ref):
    agg = agg_ref[0] + agg_ref[1]
    cnt = cnt_ref[0, :, :1] + cnt_ref[1, :, :1]
    denom = jnp.maximum(cnt, 1.0)
    m = jax.nn.relu(agg / denom + cb_ref[...])
    hn = jax.nn.relu(
        jnp.dot(h_ref[...], whh_ref[...], preferred_element_type=jnp.float32)
        + bh_ref[...]
    )
    o = jax.nn.relu(
        jnp.dot(hn, wtop_ref[...], preferred_element_type=jnp.float32)
        + jnp.dot(m, wbot_ref[...], preferred_element_type=jnp.float32)
        + bhm_ref[...]
    ) + h_ref[...]
    out_ref[...] = o
    hn_ref[...] = hn
    outr_ref[...] = jax.nn.relu(o)


def _node0(x, w, b):
    return pl.pallas_call(
        _node0_body,
        grid=(N // 1000,),
        in_specs=[
            pl.BlockSpec((1000, HID), lambda i: (i, 0)),
            pl.BlockSpec((HID, DW), lambda i: (0, 0)),
            pl.BlockSpec((1, DW), lambda i: (0, 0)),
        ],
        out_specs=pl.BlockSpec((1000, DW), lambda i: (i, 0)),
        out_shape=jax.ShapeDtypeStruct((NPAD, DW), jnp.float32),
    )(x, w, b)


def _hid(ea, w, b):
    ca = ea.shape[1]
    return pl.pallas_call(
        _hid_body,
        grid=(E // HB,),
        in_specs=[
            pl.BlockSpec((HB, ca), lambda i: (i, 0)),
            pl.BlockSpec((ca, D), lambda i: (0, 0)),
            pl.BlockSpec((1, D), lambda i: (0, 0)),
        ],
        out_specs=pl.BlockSpec((HB, D), lambda i: (i, 0)),
        out_shape=jax.ShapeDtypeStruct((EP, D), jnp.float32),
    )(ea, w, b)


def _msg(hid, xj, et_bf, bmat_p):
    return pl.pallas_call(
        _msg_body,
        grid=(EP // EB,),
        in_specs=[
            pl.BlockSpec((EB, D), lambda i: (i, 0)),
            pl.BlockSpec((EB, DW), lambda i: (i, 0)),
            pl.BlockSpec((D * D, D), lambda i: (0, 0)),
            pl.BlockSpec((DW, DW), lambda i: (0, 0)),
        ],
        out_specs=pl.BlockSpec((EB, DW), lambda i: (i, 0)),
        out_shape=jax.ShapeDtypeStruct((EP, DW), jnp.float32),
    )(hid, xj, et_bf, bmat_p)


def _update(agg2, cnt2, h, whh, bh, wtop, wbot, bhm, cb):
    return pl.pallas_call(
        _update_body,
        grid=(NPAD // NB,),
        in_specs=[
            pl.BlockSpec((2, NB, DW), lambda i: (0, i, 0)),
            pl.BlockSpec((2, NB, DW), lambda i: (0, i, 0)),
            pl.BlockSpec((NB, DW), lambda i: (i, 0)),
            pl.BlockSpec((DW, DW), lambda i: (0, 0)),
            pl.BlockSpec((1, DW), lambda i: (0, 0)),
            pl.BlockSpec((DW, DW), lambda i: (0, 0)),
            pl.BlockSpec((DW, DW), lambda i: (0, 0)),
            pl.BlockSpec((1, DW), lambda i: (0, 0)),
            pl.BlockSpec((1, DW), lambda i: (0, 0)),
        ],
        out_specs=[
            pl.BlockSpec((NB, DW), lambda i: (i, 0)),
            pl.BlockSpec((NB, DW), lambda i: (i, 0)),
            pl.BlockSpec((NB, DW), lambda i: (i, 0)),
        ],
        out_shape=[
            jax.ShapeDtypeStruct((NPAD, DW), jnp.float32),
            jax.ShapeDtypeStruct((NPAD, DW), jnp.float32),
            jax.ShapeDtypeStruct((NPAD, DW), jnp.float32),
        ],
    )(agg2, cnt2, h, whh, bh, wtop, wbot, bhm, cb)


@functools.partial(
    pl.kernel,
    mesh=_mesh,
    out_type=jax.ShapeDtypeStruct((EP, DW), jnp.float32),
    scratch_types=[
        pltpu.VMEM((CPW * LW,), jnp.int32),
        pltpu.VMEM((2, LW, DW), jnp.float32),
        pltpu.SemaphoreType.DMA,
        pltpu.SemaphoreType.DMA,
    ],
)
def _gather_sc(table_hbm, idx_hbm, out_hbm, idx_v, rows_v, sem0, sem1):
    w = lax.axis_index("c") * NTEC + lax.axis_index("s")
    base = w * CPW
    sems = (sem0, sem1)
    pltpu.sync_copy(idx_hbm.at[pl.ds(base * LW, CPW * LW)], idx_v)
    for b in range(2):
        pltpu.async_copy(
            table_hbm.at[idx_v.at[pl.ds(b * LW, LW)]], rows_v.at[b], sems[b])

    def body(i, _):
        def one(j, b):
            pltpu.make_async_copy(
                table_hbm.at[pl.ds(0, LW)], rows_v.at[b], sems[b]).wait()
            pltpu.sync_copy(rows_v.at[b], out_hbm.at[pl.ds((base + j) * LW, LW)])

            @pl.when(j + 2 < CPW)
            def _():
                pltpu.async_copy(
                    table_hbm.at[idx_v.at[pl.ds((j + 2) * LW, LW)]],
                    rows_v.at[b], sems[b])

        one(2 * i, 0)
        one(2 * i + 1, 1)
        return ()

    lax.fori_loop(0, CPW // 2, body, (), unroll=False)


@functools.partial(
    pl.kernel,
    mesh=_mesh,
    out_type=jax.ShapeDtypeStruct((NSC, NPAD, DW), jnp.float32),
    scratch_types=[
        pltpu.VMEM((CPW, LW), jnp.int32),
        pltpu.VMEM((LW, DW), jnp.float32),
        pltpu.VMEM_SHARED((NPAD, DW), jnp.float32),
    ],
)
def _scatter_sc(msg_hbm, idx_hbm, zeros_hbm, out_hbm, idx_v, row_v, agg_sh):
    c = lax.axis_index("c")
    s = lax.axis_index("s")
    base = (c * NTEC + s) * CPW
    pltpu.sync_copy(zeros_hbm.at[pl.ds(s * STRIPE, STRIPE)],
                    agg_sh.at[pl.ds(s * STRIPE, STRIPE)])

    def load_idx(i, _):
        pltpu.sync_copy(idx_hbm.at[pl.ds((base + i) * LW, LW)], idx_v.at[i])
        return ()

    lax.fori_loop(0, CPW, load_idx, (), unroll=False)
    plsc.subcore_barrier()

    def body(i, _):
        pltpu.sync_copy(msg_hbm.at[pl.ds((base + i) * LW, LW)], row_v)
        pltpu.sync_copy(row_v, agg_sh.at[idx_v.at[i]], add=True)
        return ()

    lax.fori_loop(0, CPW, body, (), unroll=False)
    plsc.subcore_barrier()
    pltpu.sync_copy(agg_sh.at[pl.ds(s * STRIPE, STRIPE)],
                    out_hbm.at[c, pl.ds(s * STRIPE, STRIPE)])


@functools.partial(
    pl.kernel,
    mesh=_mesh,
    out_type=jax.ShapeDtypeStruct((NSC, NPAD, DW), jnp.float32),
    scratch_types=[
        pltpu.VMEM((CPW, LW), jnp.int32),
        pltpu.VMEM((LW, DW), jnp.float32),
        pltpu.VMEM_SHARED((NPAD, DW), jnp.float32),
    ],
)
def _count_sc(ones_hbm, idx_hbm, zeros_hbm, out_hbm, idx_v, ones_v, cnt_sh):
    c = lax.axis_index("c")
    s = lax.axis_index("s")
    base = (c * NTEC + s) * CPW
    pltpu.sync_copy(zeros_hbm.at[pl.ds(s * STRIPE, STRIPE)],
                    cnt_sh.at[pl.ds(s * STRIPE, STRIPE)])
    pltpu.sync_copy(ones_hbm, ones_v)

    def load_idx(i, _):
        pltpu.sync_copy(idx_hbm.at[pl.ds((base + i) * LW, LW)], idx_v.at[i])
        return ()

    lax.fori_loop(0, CPW, load_idx, (), unroll=False)
    plsc.subcore_barrier()

    def body(i, _):
        pltpu.sync_copy(ones_v, cnt_sh.at[idx_v.at[i]], add=True)
        return ()

    lax.fori_loop(0, CPW, body, (), unroll=False)
    plsc.subcore_barrier()
    pltpu.sync_copy(cnt_sh.at[pl.ds(s * STRIPE, STRIPE)],
                    out_hbm.at[c, pl.ds(s * STRIPE, STRIPE)])


def kernel(x, edge_index, edge_attr, lin0_w, lin0_b, lin_h_w, lin_h_b,
           lin_hm_w, lin_hm_b, en1_w, en1_b, en2_w, en2_b, conv_b):
    src = edge_index[0]
    dst = edge_index[1]

    src_p = jnp.pad(src, (0, EP - E))
    dst_p = jnp.pad(dst, (0, EP - E), constant_values=N)
    pw = ((0, 0), (0, DW - D))
    pio = ((0, DW - D), (0, DW - D))
    lin0_w_p = jnp.pad(lin0_w, pw)
    lin0_b_p = jnp.pad(lin0_b, (0, DW - D))[None, :]
    whh_p = jnp.pad(lin_h_w, pio)
    bh_p = jnp.pad(lin_h_b, (0, DW - D))[None, :]
    wtop_p = jnp.pad(lin_hm_w[:D], pio)
    wbot_p = jnp.pad(lin_hm_w[D:], pio)
    bhm_p = jnp.pad(lin_hm_b, (0, DW - D))[None, :]
    cb_p = jnp.pad(conv_b, (0, DW - D))[None, :]
    et_bf = en2_w.T.astype(jnp.bfloat16)
    bmat_p = jnp.pad(en2_b.reshape(D, D), pio)
    zeros_nd = jnp.zeros((NPAD, DW), jnp.float32)
    ones_rows = jnp.ones((LW, DW), jnp.float32)

    hid = _hid(edge_attr, en1_w, en1_b[None, :])
    out = _node0(x, lin0_w_p, lin0_b_p)
    cnt2 = _count_sc(ones_rows, dst_p, zeros_nd)

    h = out
    for _ in range(2):
        xj = _gather_sc(out, src_p)
        msg = _msg(hid, xj, et_bf, bmat_p)
        agg2 = _scatter_sc(msg, dst_p, zeros_nd)
        out, h, outr = _update(agg2, cnt2, h, whh_p, bh_p, wtop_p, wbot_p,
                               bhm_p, cb_p)
    return outr[:N, :D]


# speedup vs baseline: 1.2071x; 1.2071x over previous
import functools

import jax
import jax.numpy as jnp
from jax import lax
from jax.experimental import pallas as pl
from jax.experimental.pallas import tpu as pltpu
from jax.experimental.pallas import tpu_sc as plsc

N = 10000
E = 160000
HID = 128
D = 64
DW = 128

NSC = 2
NTEC = 16
LW = 128
EP = 163840
CPW = EP // (NSC * NTEC * LW)
NPAD = 10240
STRIPE = NPAD // NTEC

EPH = EP // 2
CPWH = CPW // 2

EB = 1024
NB = 1280
HB = 6400

_mesh = plsc.VectorSubcoreMesh(core_axis_name="c", subcore_axis_name="s")


def _node0_body(x_ref, w_ref, b_ref, o_ref):
    o_ref[...] = jax.nn.relu(
        jnp.dot(x_ref[...], w_ref[...], preferred_element_type=jnp.float32)
        + b_ref[...]
    )


def _hid_body(ea_ref, w_ref, b_ref, o_ref):
    o_ref[...] = jax.nn.relu(
        jnp.dot(ea_ref[...], w_ref[...], preferred_element_type=jnp.float32)
        + b_ref[...]
    )


def _msg_body(hid_ref, xj_ref, et_ref, bmat_ref, o_ref):
    hbT = hid_ref[...].T.astype(jnp.bfloat16)
    wft = jnp.dot(et_ref[...], hbT, preferred_element_type=jnp.float32)
    xjT = xj_ref[...].T
    acc = xjT[0:1, :] * wft[0:D, :]
    for i in range(1, D):
        acc = acc + xjT[i:i + 1, :] * wft[i * D:(i + 1) * D, :]
    msg64 = acc.T
    o_ref[...] = jnp.concatenate(
        [msg64, jnp.zeros_like(msg64)], axis=1
    ) + jnp.dot(xj_ref[...], bmat_ref[...], preferred_element_type=jnp.float32)


def _update_body(agg_ref, aggb_ref, cnt_ref, h_ref, whh_ref, bh_ref, wtop_ref,
                 wbot_ref, bhm_ref, cb_ref, out_ref, hn_ref, outr_ref):
    agg = (agg_ref[0] + agg_ref[1]) + (aggb_ref[0] + aggb_ref[1])
    cnt = cnt_ref[0, :, :1] + cnt_ref[1, :, :1]
    denom = jnp.maximum(cnt, 1.0)
    m = jax.nn.relu(agg / denom + cb_ref[...])
    hn = jax.nn.relu(
        jnp.dot(h_ref[...], whh_ref[...], preferred_element_type=jnp.float32)
        + bh_ref[...]
    )
    o = jax.nn.relu(
        jnp.dot(hn, wtop_ref[...], preferred_element_type=jnp.float32)
        + jnp.dot(m, wbot_ref[...], preferred_element_type=jnp.float32)
        + bhm_ref[...]
    ) + h_ref[...]
    out_ref[...] = o
    hn_ref[...] = hn
    outr_ref[...] = jax.nn.relu(o)


def _node0(x, w, b):
    return pl.pallas_call(
        _node0_body,
        grid=(N // 1000,),
        in_specs=[
            pl.BlockSpec((1000, HID), lambda i: (i, 0)),
            pl.BlockSpec((HID, DW), lambda i: (0, 0)),
            pl.BlockSpec((1, DW), lambda i: (0, 0)),
        ],
        out_specs=pl.BlockSpec((1000, DW), lambda i: (i, 0)),
        out_shape=jax.ShapeDtypeStruct((NPAD, DW), jnp.float32),
    )(x, w, b)


def _hid(ea, w, b):
    ca = ea.shape[1]
    return pl.pallas_call(
        _hid_body,
        grid=(E // HB,),
        in_specs=[
            pl.BlockSpec((HB, ca), lambda i: (i, 0)),
            pl.BlockSpec((ca, D), lambda i: (0, 0)),
            pl.BlockSpec((1, D), lambda i: (0, 0)),
        ],
        out_specs=pl.BlockSpec((HB, D), lambda i: (i, 0)),
        out_shape=jax.ShapeDtypeStruct((EP, D), jnp.float32),
    )(ea, w, b)


def _msg_half(half):
    off = half * (EPH // EB)

    def call(hid, xj, et_bf, bmat_p):
        return pl.pallas_call(
            _msg_body,
            grid=(EPH // EB,),
            in_specs=[
                pl.BlockSpec((EB, D), lambda i: (i + off, 0)),
                pl.BlockSpec((EB, DW), lambda i: (i, 0)),
                pl.BlockSpec((D * D, D), lambda i: (0, 0)),
                pl.BlockSpec((DW, DW), lambda i: (0, 0)),
            ],
            out_specs=pl.BlockSpec((EB, DW), lambda i: (i, 0)),
            out_shape=jax.ShapeDtypeStruct((EPH, DW), jnp.float32),
        )(hid, xj, et_bf, bmat_p)

    return call


def _update(agg2, agg2b, cnt2, h, whh, bh, wtop, wbot, bhm, cb):
    return pl.pallas_call(
        _update_body,
        grid=(NPAD // NB,),
        in_specs=[
            pl.BlockSpec((2, NB, DW), lambda i: (0, i, 0)),
            pl.BlockSpec((2, NB, DW), lambda i: (0, i, 0)),
            pl.BlockSpec((2, NB, DW), lambda i: (0, i, 0)),
            pl.BlockSpec((NB, DW), lambda i: (i, 0)),
            pl.BlockSpec((DW, DW), lambda i: (0, 0)),
            pl.BlockSpec((1, DW), lambda i: (0, 0)),
            pl.BlockSpec((DW, DW), lambda i: (0, 0)),
            pl.BlockSpec((DW, DW), lambda i: (0, 0)),
            pl.BlockSpec((1, DW), lambda i: (0, 0)),
            pl.BlockSpec((1, DW), lambda i: (0, 0)),
        ],
        out_specs=[
            pl.BlockSpec((NB, DW), lambda i: (i, 0)),
            pl.BlockSpec((NB, DW), lambda i: (i, 0)),
            pl.BlockSpec((NB, DW), lambda i: (i, 0)),
        ],
        out_shape=[
            jax.ShapeDtypeStruct((NPAD, DW), jnp.float32),
            jax.ShapeDtypeStruct((NPAD, DW), jnp.float32),
            jax.ShapeDtypeStruct((NPAD, DW), jnp.float32),
        ],
    )(agg2, agg2b, cnt2, h, whh, bh, wtop, wbot, bhm, cb)


def _make_gather(cpw):
    @functools.partial(
        pl.kernel,
        mesh=_mesh,
        out_type=jax.ShapeDtypeStruct((cpw * 32 * LW, DW), jnp.float32),
        scratch_types=[
            pltpu.VMEM((cpw * LW,), jnp.int32),
            pltpu.VMEM((2, LW, DW), jnp.float32),
            pltpu.SemaphoreType.DMA,
            pltpu.SemaphoreType.DMA,
        ],
    )
    def gather(table_hbm, idx_hbm, out_hbm, idx_v, rows_v, sem0, sem1):
        w = lax.axis_index("c") * NTEC + lax.axis_index("s")
        base = w * cpw
        sems = (sem0, sem1)
        pltpu.sync_copy(idx_hbm.at[pl.ds(base * LW, cpw * LW)], idx_v)
        for b in range(2):
            pltpu.async_copy(
                table_hbm.at[idx_v.at[pl.ds(b * LW, LW)]], rows_v.at[b],
                sems[b])

        def body(i, _):
            def one(j, b):
                pltpu.make_async_copy(
                    table_hbm.at[pl.ds(0, LW)], rows_v.at[b], sems[b]).wait()
                pltpu.sync_copy(rows_v.at[b],
                                out_hbm.at[pl.ds((base + j) * LW, LW)])

                @pl.when(j + 2 < cpw)
                def _():
                    pltpu.async_copy(
                        table_hbm.at[idx_v.at[pl.ds((j + 2) * LW, LW)]],
                        rows_v.at[b], sems[b])

            one(2 * i, 0)
            one(2 * i + 1, 1)
            return ()

        lax.fori_loop(0, cpw // 2, body, (), unroll=False)

    return gather


def _make_scatter(cpw):
    @functools.partial(
        pl.kernel,
        mesh=_mesh,
        out_type=jax.ShapeDtypeStruct((NSC, NPAD, DW), jnp.float32),
        scratch_types=[
            pltpu.VMEM((cpw, LW), jnp.int32),
            pltpu.VMEM((LW, DW), jnp.float32),
            pltpu.VMEM_SHARED((NPAD, DW), jnp.float32),
        ],
    )
    def scatter(msg_hbm, idx_hbm, zeros_hbm, out_hbm, idx_v, row_v, agg_sh):
        c = lax.axis_index("c")
        s = lax.axis_index("s")
        base = (c * NTEC + s) * cpw
        pltpu.sync_copy(zeros_hbm.at[pl.ds(s * STRIPE, STRIPE)],
                        agg_sh.at[pl.ds(s * STRIPE, STRIPE)])

        def load_idx(i, _):
            pltpu.sync_copy(idx_hbm.at[pl.ds((base + i) * LW, LW)],
                            idx_v.at[i])
            return ()

        lax.fori_loop(0, cpw, load_idx, (), unroll=False)
        plsc.subcore_barrier()

        def body(i, _):
            pltpu.sync_copy(msg_hbm.at[pl.ds((base + i) * LW, LW)], row_v)
            pltpu.sync_copy(row_v, agg_sh.at[idx_v.at[i]], add=True)
            return ()

        lax.fori_loop(0, cpw, body, (), unroll=False)
        plsc.subcore_barrier()
        pltpu.sync_copy(agg_sh.at[pl.ds(s * STRIPE, STRIPE)],
                        out_hbm.at[c, pl.ds(s * STRIPE, STRIPE)])

    return scatter


_gather_h = _make_gather(CPWH)
_scatter_h = _make_scatter(CPWH)
_msg_a = _msg_half(0)
_msg_b = _msg_half(1)


@functools.partial(
    pl.kernel,
    mesh=_mesh,
    out_type=jax.ShapeDtypeStruct((NSC, NPAD, DW), jnp.float32),
    scratch_types=[
        pltpu.VMEM((CPW, LW), jnp.int32),
        pltpu.VMEM((LW, DW), jnp.float32),
        pltpu.VMEM_SHARED((NPAD, DW), jnp.float32),
    ],
)
def _count_sc(ones_hbm, idx_hbm, zeros_hbm, out_hbm, idx_v, ones_v, cnt_sh):
    c = lax.axis_index("c")
    s = lax.axis_index("s")
    base = (c * NTEC + s) * CPW
    pltpu.sync_copy(zeros_hbm.at[pl.ds(s * STRIPE, STRIPE)],
                    cnt_sh.at[pl.ds(s * STRIPE, STRIPE)])
    pltpu.sync_copy(ones_hbm, ones_v)

    def load_idx(i, _):
        pltpu.sync_copy(idx_hbm.at[pl.ds((base + i) * LW, LW)], idx_v.at[i])
        return ()

    lax.fori_loop(0, CPW, load_idx, (), unroll=False)
    plsc.subcore_barrier()

    def body(i, _):
        pltpu.sync_copy(ones_v, cnt_sh.at[idx_v.at[i]], add=True)
        return ()

    lax.fori_loop(0, CPW, body, (), unroll=False)
    plsc.subcore_barrier()
    pltpu.sync_copy(cnt_sh.at[pl.ds(s * STRIPE, STRIPE)],
                    out_hbm.at[c, pl.ds(s * STRIPE, STRIPE)])


def kernel(x, edge_index, edge_attr, lin0_w, lin0_b, lin_h_w, lin_h_b,
           lin_hm_w, lin_hm_b, en1_w, en1_b, en2_w, en2_b, conv_b):
    src = edge_index[0]
    dst = edge_index[1]

    src_p = jnp.pad(src, (0, EP - E))
    dst_p = jnp.pad(dst, (0, EP - E), constant_values=N)
    pw = ((0, 0), (0, DW - D))
    pio = ((0, DW - D), (0, DW - D))
    lin0_w_p = jnp.pad(lin0_w, pw)
    lin0_b_p = jnp.pad(lin0_b, (0, DW - D))[None, :]
    whh_p = jnp.pad(lin_h_w, pio)
    bh_p = jnp.pad(lin_h_b, (0, DW - D))[None, :]
    wtop_p = jnp.pad(lin_hm_w[:D], pio)
    wbot_p = jnp.pad(lin_hm_w[D:], pio)
    bhm_p = jnp.pad(lin_hm_b, (0, DW - D))[None, :]
    cb_p = jnp.pad(conv_b, (0, DW - D))[None, :]
    et_bf = en2_w.T.astype(jnp.bfloat16)
    bmat_p = jnp.pad(en2_b.reshape(D, D), pio)
    zeros_nd = jnp.zeros((NPAD, DW), jnp.float32)
    ones_rows = jnp.ones((LW, DW), jnp.float32)

    hid = _hid(edge_attr, en1_w, en1_b[None, :])
    out = _node0(x, lin0_w_p, lin0_b_p)
    cnt2 = _count_sc(ones_rows, dst_p, zeros_nd)

    src_a, src_b = src_p[:EPH], src_p[EPH:]
    dst_a, dst_b = dst_p[:EPH], dst_p[EPH:]
    h = out
    for _ in range(2):
        xja = _gather_h(out, src_a)
        msga = _msg_a(hid, xja, et_bf, bmat_p)
        xjb = _gather_h(out, src_b)
        msgb = _msg_b(hid, xjb, et_bf, bmat_p)
        agga = _scatter_h(msga, dst_a, zeros_nd)
        aggb = _scatter_h(msgb, dst_b, zeros_nd)
        out, h, outr = _update(agga, aggb, cnt2, h, whh_p, bh_p, wtop_p,
                               wbot_p, bhm_p, cb_p)
    return outr[:N, :D]
